# baseline (speedup 1.0000x reference)
import jax
import jax.numpy as jnp
from jax import lax
from jax.experimental import pallas as pl
from jax.experimental.pallas import tpu as pltpu
from jax.experimental.pallas import tpu_sc as plsc

N_NODES = 10000
N_EDGES = 160000
D_FEAT = 256

NUM_CORES = 2
NUM_SUBCORES = 16
NUM_WORKERS = NUM_CORES * NUM_SUBCORES

EDGES_PER_WORKER = N_EDGES // NUM_WORKERS
BLOCK_E = 200
LANES = 16
NUM_GROUPS = (BLOCK_E + LANES - 1) // LANES
BLOCK_PAD = NUM_GROUPS * LANES
NUM_BLOCKS = EDGES_PER_WORKER // BLOCK_E
D_WORDS = D_FEAT // 2
D_PAIRS = D_WORDS // LANES


def _permute(v, perm):
    return lax.gather(
        v, perm.reshape(LANES, 1),
        lax.GatherDimensionNumbers(
            offset_dims=(), collapsed_slice_dims=(0,), start_index_map=(0,)),
        slice_sizes=(1,),
        mode=lax.GatherScatterMode.PROMISE_IN_BOUNDS)


def _sc_kernel_body(x_hbm, src_hbm, dst_hbm, out_hbm,
                    idx_s0, idx_d0, idx_s1, idx_d1,
                    rows_s0, rows_d0, rows_s1, rows_d1, scores,
                    sem_s0, sem_d0, sem_s1, sem_d1):
    wid = lax.axis_index("s") * NUM_CORES + lax.axis_index("c")
    worker_base = wid * EDGES_PER_WORKER
    lane_iota = lax.iota(jnp.int32, LANES)
    zeros_f32 = (lane_iota - lane_iota).astype(jnp.float32)

    idx_bufs = ((idx_s0, idx_d0), (idx_s1, idx_d1))
    row_bufs = ((rows_s0, rows_d0), (rows_s1, rows_d1))
    sems = ((sem_s0, sem_d0), (sem_s1, sem_d1))

    def issue(blk, slot):
        base = worker_base + blk * BLOCK_E
        idx_s, idx_d = idx_bufs[slot]
        rows_s, rows_d = row_bufs[slot]
        sem_s, sem_d = sems[slot]
        pltpu.sync_copy(src_hbm.at[pl.ds(base, BLOCK_PAD)], idx_s)
        pltpu.sync_copy(dst_hbm.at[pl.ds(base, BLOCK_PAD)], idx_d)
        pltpu.async_copy(x_hbm.at[idx_s], rows_s, sem_s)
        pltpu.async_copy(x_hbm.at[idx_d], rows_d, sem_d)

    def wait(slot):
        idx_s, idx_d = idx_bufs[slot]
        rows_s, rows_d = row_bufs[slot]
        sem_s, sem_d = sems[slot]
        pltpu.make_async_copy(x_hbm.at[idx_s], rows_s, sem_s).wait()
        pltpu.make_async_copy(x_hbm.at[idx_d], rows_d, sem_d).wait()

    def compute_and_flush(blk, slot):
        base = worker_base + blk * BLOCK_E
        rows_s, rows_d = row_bufs[slot]

        def group_body(g, carry):
            row0 = g * LANES
            accs = []
            for e in range(LANES):
                row = row0 + e
                acc = zeros_f32
                for p in range(D_PAIRS):
                    ws = rows_s[row, pl.ds(p * LANES, LANES)]
                    wd = rows_d[row, pl.ds(p * LANES, LANES)]
                    s_lo = lax.bitcast_convert_type(ws << 16, jnp.float32)
                    d_lo = lax.bitcast_convert_type(wd << 16, jnp.float32)
                    s_hi = lax.bitcast_convert_type(ws, jnp.float32)
                    d_hi = lax.bitcast_convert_type(wd, jnp.float32)
                    acc = acc + s_lo * d_lo + s_hi * d_hi
                accs.append(acc)
            for m in (1, 2, 4, 8):
                mask = (lane_iota & m) == 0
                nxt = []
                for i in range(0, len(accs), 2):
                    a, b = accs[i], accs[i + 1]
                    sa = a + _permute(a, lane_iota ^ m)
                    sb = b + _permute(b, lane_iota ^ m)
                    nxt.append(jnp.where(mask, sa, sb))
                accs = nxt
            scores[pl.ds(row0, LANES)] = accs[0]
            return carry

        lax.fori_loop(0, NUM_GROUPS, group_body, 0)
        pltpu.sync_copy(scores.at[pl.ds(0, BLOCK_E)],
                        out_hbm.at[pl.ds(base, BLOCK_E)])

    issue(0, 0)

    def pair_body(i, carry):
        blk0 = i * 2
        wait(0)
        issue(blk0 + 1, 1)
        compute_and_flush(blk0, 0)
        wait(1)
        issue(blk0 + 2, 0)
        compute_and_flush(blk0 + 1, 1)
        return carry

    lax.fori_loop(0, (NUM_BLOCKS - 1) // 2, pair_body, 0)
    wait(0)
    compute_and_flush(NUM_BLOCKS - 1, 0)


def kernel(x, edge_index):
    u = lax.bitcast_convert_type(x, jnp.uint32)
    t = u + jnp.uint32(0x7FFF) + ((u >> 16) & jnp.uint32(1))
    xw = lax.bitcast_convert_type(
        (t[:, D_WORDS:] & jnp.uint32(0xFFFF0000)) | (t[:, :D_WORDS] >> 16),
        jnp.int32)
    pad = jnp.zeros((BLOCK_PAD - BLOCK_E,), jnp.int32)
    src = jnp.concatenate([edge_index[0], pad])
    dst = jnp.concatenate([edge_index[1], pad])

    mesh = plsc.VectorSubcoreMesh(core_axis_name="c", subcore_axis_name="s")
    run = pl.kernel(
        _sc_kernel_body,
        mesh=mesh,
        out_type=jax.ShapeDtypeStruct((N_EDGES,), jnp.float32),
        scratch_types=[
            pltpu.VMEM((BLOCK_PAD,), jnp.int32),
            pltpu.VMEM((BLOCK_PAD,), jnp.int32),
            pltpu.VMEM((BLOCK_PAD,), jnp.int32),
            pltpu.VMEM((BLOCK_PAD,), jnp.int32),
            pltpu.VMEM((BLOCK_PAD, D_WORDS), jnp.int32),
            pltpu.VMEM((BLOCK_PAD, D_WORDS), jnp.int32),
            pltpu.VMEM((BLOCK_PAD, D_WORDS), jnp.int32),
            pltpu.VMEM((BLOCK_PAD, D_WORDS), jnp.int32),
            pltpu.VMEM((BLOCK_PAD,), jnp.float32),
            pltpu.SemaphoreType.DMA,
            pltpu.SemaphoreType.DMA,
            pltpu.SemaphoreType.DMA,
            pltpu.SemaphoreType.DMA,
        ],
    )
    score = run(xw, src, dst)
    return score.reshape(N_EDGES, 1)


# speedup vs baseline: 1.2782x; 1.2782x over previous
import jax
import jax.numpy as jnp
from jax import lax
from jax.experimental import pallas as pl
from jax.experimental.pallas import tpu as pltpu
from jax.experimental.pallas import tpu_sc as plsc

N_NODES = 10000
N_EDGES = 160000
D_FEAT = 256

NUM_CORES = 2
NUM_SUBCORES = 16
NUM_WORKERS = NUM_CORES * NUM_SUBCORES

EDGES_PER_WORKER = N_EDGES // NUM_WORKERS
BLOCK_E = 200
LANES = 16
NUM_GROUPS = (BLOCK_E + LANES - 1) // LANES
BLOCK_PAD = NUM_GROUPS * LANES
NUM_BLOCKS = EDGES_PER_WORKER // BLOCK_E
D_WORDS = D_FEAT // 2
D_PAIRS = D_WORDS // LANES


def _permute(v, perm):
    return lax.gather(
        v, perm.reshape(LANES, 1),
        lax.GatherDimensionNumbers(
            offset_dims=(), collapsed_slice_dims=(0,), start_index_map=(0,)),
        slice_sizes=(1,),
        mode=lax.GatherScatterMode.PROMISE_IN_BOUNDS)


def _sc_kernel_body(x_hbm, src_hbm, dst_hbm, out_hbm,
                    idx_s0, idx_d0, idx_s1, idx_d1,
                    rows_s0, rows_d0, rows_s1, rows_d1, scores,
                    sem_s0, sem_d0, sem_s1, sem_d1, sem_i0, sem_i1):
    wid = lax.axis_index("s") * NUM_CORES + lax.axis_index("c")
    worker_base = wid * EDGES_PER_WORKER
    lane_iota = lax.iota(jnp.int32, LANES)
    zeros_f32 = (lane_iota - lane_iota).astype(jnp.float32)

    idx_bufs = ((idx_s0, idx_d0), (idx_s1, idx_d1))
    row_bufs = ((rows_s0, rows_d0), (rows_s1, rows_d1))
    sems = ((sem_s0, sem_d0), (sem_s1, sem_d1))
    idx_sems = (sem_i0, sem_i1)

    def copy_idx(blk, slot):
        base = worker_base + jnp.minimum(blk, NUM_BLOCKS - 1) * BLOCK_E
        idx_s, idx_d = idx_bufs[slot]
        sem_i = idx_sems[slot]
        pltpu.async_copy(src_hbm.at[pl.ds(base, BLOCK_PAD)], idx_s, sem_i)
        pltpu.async_copy(dst_hbm.at[pl.ds(base, BLOCK_PAD)], idx_d, sem_i)

    def start_gather(slot):
        idx_s, idx_d = idx_bufs[slot]
        rows_s, rows_d = row_bufs[slot]
        sem_s, sem_d = sems[slot]
        sem_i = idx_sems[slot]
        pltpu.make_async_copy(src_hbm.at[pl.ds(0, BLOCK_PAD)], idx_s,
                              sem_i).wait()
        pltpu.make_async_copy(dst_hbm.at[pl.ds(0, BLOCK_PAD)], idx_d,
                              sem_i).wait()
        pltpu.async_copy(x_hbm.at[idx_s], rows_s, sem_s)
        pltpu.async_copy(x_hbm.at[idx_d], rows_d, sem_d)

    def wait(slot):
        idx_s, idx_d = idx_bufs[slot]
        rows_s, rows_d = row_bufs[slot]
        sem_s, sem_d = sems[slot]
        pltpu.make_async_copy(x_hbm.at[idx_s], rows_s, sem_s).wait()
        pltpu.make_async_copy(x_hbm.at[idx_d], rows_d, sem_d).wait()

    def compute_and_flush(blk, slot):
        base = worker_base + blk * BLOCK_E
        rows_s, rows_d = row_bufs[slot]

        def group_body(g, carry):
            row0 = g * LANES
            accs = []
            for e in range(LANES):
                row = row0 + e
                acc = zeros_f32
                for p in range(D_PAIRS):
                    ws = rows_s[row, pl.ds(p * LANES, LANES)]
                    wd = rows_d[row, pl.ds(p * LANES, LANES)]
                    s_lo = lax.bitcast_convert_type(ws << 16, jnp.float32)
                    d_lo = lax.bitcast_convert_type(wd << 16, jnp.float32)
                    s_hi = lax.bitcast_convert_type(ws, jnp.float32)
                    d_hi = lax.bitcast_convert_type(wd, jnp.float32)
                    acc = acc + s_lo * d_lo + s_hi * d_hi
                accs.append(acc)
            for m in (1, 2, 4, 8):
                mask = (lane_iota & m) == 0
                nxt = []
                for i in range(0, len(accs), 2):
                    a, b = accs[i], accs[i + 1]
                    sa = a + _permute(a, lane_iota ^ m)
                    sb = b + _permute(b, lane_iota ^ m)
                    nxt.append(jnp.where(mask, sa, sb))
                accs = nxt
            scores[pl.ds(row0, LANES)] = accs[0]
            return carry

        lax.fori_loop(0, NUM_GROUPS, group_body, 0)
        pltpu.sync_copy(scores.at[pl.ds(0, BLOCK_E)],
                        out_hbm.at[pl.ds(base, BLOCK_E)])

    copy_idx(0, 0)
    start_gather(0)
    copy_idx(1, 1)

    def pair_body(i, carry):
        blk0 = i * 2
        wait(0)
        start_gather(1)
        copy_idx(blk0 + 2, 0)
        compute_and_flush(blk0, 0)
        wait(1)
        start_gather(0)
        copy_idx(blk0 + 3, 1)
        compute_and_flush(blk0 + 1, 1)
        return carry

    lax.fori_loop(0, (NUM_BLOCKS - 1) // 2, pair_body, 0)
    wait(0)
    compute_and_flush(NUM_BLOCKS - 1, 0)


def kernel(x, edge_index):
    u = lax.bitcast_convert_type(x, jnp.uint32)
    t = u + jnp.uint32(0x7FFF) + ((u >> 16) & jnp.uint32(1))
    xw = lax.bitcast_convert_type(
        (t[:, D_WORDS:] & jnp.uint32(0xFFFF0000)) | (t[:, :D_WORDS] >> 16),
        jnp.int32)
    pad = jnp.zeros((BLOCK_PAD - BLOCK_E,), jnp.int32)
    src = jnp.concatenate([edge_index[0], pad])
    dst = jnp.concatenate([edge_index[1], pad])

    mesh = plsc.VectorSubcoreMesh(core_axis_name="c", subcore_axis_name="s")
    run = pl.kernel(
        _sc_kernel_body,
        mesh=mesh,
        out_type=jax.ShapeDtypeStruct((N_EDGES,), jnp.float32),
        scratch_types=[
            pltpu.VMEM((BLOCK_PAD,), jnp.int32),
            pltpu.VMEM((BLOCK_PAD,), jnp.int32),
            pltpu.VMEM((BLOCK_PAD,), jnp.int32),
            pltpu.VMEM((BLOCK_PAD,), jnp.int32),
            pltpu.VMEM((BLOCK_PAD, D_WORDS), jnp.int32),
            pltpu.VMEM((BLOCK_PAD, D_WORDS), jnp.int32),
            pltpu.VMEM((BLOCK_PAD, D_WORDS), jnp.int32),
            pltpu.VMEM((BLOCK_PAD, D_WORDS), jnp.int32),
            pltpu.VMEM((BLOCK_PAD,), jnp.float32),
            pltpu.SemaphoreType.DMA,
            pltpu.SemaphoreType.DMA,
            pltpu.SemaphoreType.DMA,
            pltpu.SemaphoreType.DMA,
            pltpu.SemaphoreType.DMA,
            pltpu.SemaphoreType.DMA,
        ],
    )
    score = run(xw, src, dst)
    return score.reshape(N_EDGES, 1)
